# baseline (speedup 1.0000x reference)
import functools

import jax
import jax.numpy as jnp
from jax import lax
from jax.experimental import pallas as pl
from jax.experimental.pallas import tpu as pltpu
from jax.experimental.pallas import tpu_sc as plsc

N = 10000
E = 320000
D = 128
H = 128
NPAD = 10240
NC = 2
NS = 16
NW = NC * NS
CH = 128
NCHUNK = 80
EPW = NCHUNK * CH
EP = NW * EPW
NBUF = 4
GROUPS = NCHUNK // NBUF
RPW = NPAD // NS


def _mesh():
    return plsc.VectorSubcoreMesh(
        core_axis_name="c", subcore_axis_name="s",
        num_cores=NC, num_subcores=NS)


def _cparams():
    return pltpu.CompilerParams(needs_layout_passes=False)


def _worker():
    cid = lax.axis_index("c")
    sid = lax.axis_index("s")
    return cid, sid, cid * NS + sid


@functools.cache
def _make_deg():
    @functools.partial(
        pl.kernel,
        out_type=jax.ShapeDtypeStruct((NC, NPAD), jnp.float32),
        mesh=_mesh(),
        compiler_params=_cparams(),
        scratch_types=[
            pltpu.VMEM((NCHUNK, CH), jnp.int32),
            pltpu.VMEM((NPAD,), jnp.float32),
            pltpu.VMEM((NS, RPW), jnp.float32),
            pltpu.VMEM((RPW,), jnp.float32),
            pltpu.VMEM_SHARED((NS, NPAD), jnp.float32),
        ],
    )
    def deg_kernel(dst_hbm, out_hbm, idx2, acc_t, red, out_v, part_sh):
        cid, sid, wid = _worker()
        pltpu.sync_copy(dst_hbm.at[wid], idx2)

        @pl.loop(0, NPAD // 16)
        def _zero(i):
            acc_t[pl.ds(i * 16, 16)] = jnp.zeros((16,), jnp.float32)

        ones = jnp.ones((16,), jnp.float32)

        @pl.loop(0, NCHUNK)
        def _chunks(j):
            @pl.loop(0, CH // 16)
            def _vecs(k):
                idxv = idx2[j, pl.ds(k * 16, 16)]
                plsc.addupdate_scatter(acc_t, [idxv], ones)

        pltpu.sync_copy(acc_t, part_sh.at[sid])
        plsc.subcore_barrier()
        for s in range(NS):
            pltpu.sync_copy(part_sh.at[s, pl.ds(sid * RPW, RPW)], red.at[s])

        @pl.loop(0, RPW // 16)
        def _red(i):
            v = jnp.zeros((16,), jnp.float32)
            for s in range(NS):
                v = v + red[s, pl.ds(i * 16, 16)]
            out_v[pl.ds(i * 16, 16)] = v

        pltpu.sync_copy(out_v, out_hbm.at[cid, pl.ds(sid * RPW, RPW)])

    return deg_kernel


@functools.cache
def _make_prop():
    @functools.partial(
        pl.kernel,
        out_type=jax.ShapeDtypeStruct((NC, NPAD, H), jnp.float32),
        mesh=_mesh(),
        compiler_params=_cparams(),
        scratch_types=[
            pltpu.VMEM((2, CH), jnp.int32),
            pltpu.VMEM((2, CH), jnp.int32),
            pltpu.VMEM((2, CH, H), jnp.float32),
            pltpu.VMEM_SHARED((NPAD, H), jnp.float32),
            pltpu.SemaphoreType.DMA((2,)),
            pltpu.SemaphoreType.DMA((2,)),
        ],
    )
    def prop_kernel(y_hbm, src_hbm, dst_hbm, zeros_hbm, out_hbm,
                    idx_s, idx_d, rows2, acc_sh, sem_i, sem_r):
        cid, sid, wid = _worker()
        pltpu.sync_copy(zeros_hbm.at[pl.ds(sid * RPW, RPW), :],
                        acc_sh.at[pl.ds(sid * RPW, RPW), :])
        pltpu.sync_copy(src_hbm.at[wid, 0], idx_s.at[0])
        pltpu.sync_copy(dst_hbm.at[wid, 0], idx_d.at[0])
        plsc.subcore_barrier()
        pltpu.async_copy(y_hbm.at[idx_s.at[0]], rows2.at[0], sem_r.at[0])

        @pl.loop(0, NCHUNK)
        def _chunks(j):
            par = lax.rem(j, 2)
            opar = 1 - par
            nxt = jnp.minimum(j + 1, NCHUNK - 1)
            pltpu.async_copy(src_hbm.at[wid, nxt], idx_s.at[opar],
                             sem_i.at[opar])
            pltpu.async_copy(dst_hbm.at[wid, nxt], idx_d.at[opar],
                             sem_i.at[opar])
            pltpu.make_async_copy(
                y_hbm.at[idx_s.at[0]], rows2.at[par], sem_r.at[par]).wait()
            pltpu.sync_copy(rows2.at[par], acc_sh.at[idx_d.at[par]],
                            add=True)
            pltpu.make_async_copy(src_hbm.at[wid, 0], idx_s.at[opar],
                                  sem_i.at[opar]).wait()
            pltpu.make_async_copy(dst_hbm.at[wid, 0], idx_d.at[opar],
                                  sem_i.at[opar]).wait()
            pltpu.async_copy(y_hbm.at[idx_s.at[opar]], rows2.at[opar],
                             sem_r.at[opar])

        pltpu.make_async_copy(
            y_hbm.at[idx_s.at[0]], rows2.at[0], sem_r.at[0]).wait()

        plsc.subcore_barrier()
        pltpu.sync_copy(acc_sh.at[pl.ds(sid * RPW, RPW), :],
                        out_hbm.at[cid, pl.ds(sid * RPW, RPW), :])

    return prop_kernel


@functools.cache
def _make_struct():
    @functools.partial(
        pl.kernel,
        out_type=jax.ShapeDtypeStruct((NW, (NCHUNK + 1) * CH), jnp.float32),
        mesh=_mesh(),
        compiler_params=_cparams(),
        scratch_types=[
            pltpu.VMEM((NCHUNK, CH), jnp.int32),
            pltpu.VMEM((NCHUNK, CH), jnp.int32),
        ] + [pltpu.VMEM((CH, H), jnp.float32)] * 4
          + [
            pltpu.VMEM((H,), jnp.float32),
            pltpu.VMEM((16,), jnp.float32),
            pltpu.VMEM((CH,), jnp.float32),
            pltpu.VMEM((CH,), jnp.float32),
            pltpu.VMEM((256,), jnp.float32),
        ] + [pltpu.SemaphoreType.DMA] * 6,
    )
    def struct_kernel(a_hbm, b_hbm, src_hbm, dst_hbm, w2_hbm, sb2_hbm,
                      out_hbm, idx_s, idx_d, ra0, rb0, ra1, rb1, w2_v,
                      sb2_v, dv0, dv1, tbuf, ga0, gb0, ga1, gb1, so0, so1):
        rows_a = (ra0, ra1)
        rows_b = (rb0, rb1)
        dot_v = (dv0, dv1)
        sem_a = (ga0, ga1)
        sem_b = (gb0, gb1)
        sem_o = (so0, so1)
        cid, sid, wid = _worker()
        pltpu.sync_copy(w2_hbm, w2_v)
        pltpu.sync_copy(sb2_hbm, sb2_v)
        pltpu.sync_copy(src_hbm.at[wid], idx_s)
        pltpu.sync_copy(dst_hbm.at[wid], idx_d)

        def compute_chunk(b):
            pltpu.make_async_copy(
                a_hbm.at[idx_s.at[0]], rows_a[b], sem_a[b]).wait()
            pltpu.make_async_copy(
                b_hbm.at[idx_d.at[0]], rows_b[b], sem_b[b]).wait()
            pltpu.make_async_copy(
                dot_v[b], out_hbm.at[wid, pl.ds(0, CH)], sem_o[b]).wait()

            @pl.loop(0, CH // 16)
            def _groups(g):
                for e16 in range(16):
                    e = g * 16 + e16
                    acc = jnp.zeros((16,), jnp.float32)
                    for c in range(H // 16):
                        va = rows_a[b][e, pl.ds(c * 16, 16)]
                        vb = rows_b[b][e, pl.ds(c * 16, 16)]
                        t = jnp.maximum(va + vb, 0.0)
                        acc = acc + t * w2_v[pl.ds(c * 16, 16)]
                    tbuf[pl.ds(e16 * 16, 16)] = acc
                flat = lax.iota(jnp.int32, 16) * 16
                vsum = jnp.zeros((16,), jnp.float32)
                for c in range(16):
                    vsum = vsum + plsc.load_gather(tbuf, [flat + c])
                t = vsum + sb2_v[...]
                dot_v[b][pl.ds(g * 16, 16)] = 1.0 / (1.0 + jnp.exp(-t))

        def store_chunk(b, j):
            pltpu.async_copy(dot_v[b], out_hbm.at[wid, pl.ds(j * CH, CH)],
                             sem_o[b])

        def refill(b, j):
            pltpu.async_copy(a_hbm.at[idx_s.at[j]], rows_a[b], sem_a[b])
            pltpu.async_copy(b_hbm.at[idx_d.at[j]], rows_b[b], sem_b[b])

        for b in range(2):
            refill(b, b)
            pltpu.async_copy(dot_v[b],
                             out_hbm.at[wid, pl.ds(NCHUNK * CH, CH)],
                             sem_o[b])

        @pl.loop(0, NCHUNK // 2 - 1)
        def _pairs(p):
            for b in range(2):
                j = p * 2 + b
                compute_chunk(b)
                store_chunk(b, j)
                refill(b, j + 2)

        for b in range(2):
            compute_chunk(b)
            store_chunk(b, NCHUNK - 2 + b)

        for b in range(2):
            pltpu.make_async_copy(
                dot_v[b], out_hbm.at[wid, pl.ds(0, CH)], sem_o[b]).wait()

    return struct_kernel


def _mm1_body(x_ref, w_ref, xw_ref):
    xw_ref[...] = jnp.dot(x_ref[...], w_ref[...],
                          preferred_element_type=jnp.float32,
                          precision=lax.Precision.HIGHEST)


def _mm1(xp, W1):
    R = 1024
    return pl.pallas_call(
        _mm1_body,
        grid=(NPAD // R,),
        in_specs=[
            pl.BlockSpec((R, D), lambda i: (i, 0)),
            pl.BlockSpec((D, H), lambda i: (0, 0)),
        ],
        out_specs=pl.BlockSpec((R, H), lambda i: (i, 0)),
        out_shape=jax.ShapeDtypeStruct((NPAD, H), jnp.float32),
    )(xp, W1)


def _scale_body(xw_ref, deg_ref, y_ref, dis_ref):
    dis = lax.rsqrt(deg_ref[...])
    y_ref[...] = xw_ref[...] * dis
    dis_ref[...] = dis


def _scale1(xw, deg2):
    R = 1024
    return pl.pallas_call(
        _scale_body,
        grid=(NPAD // R,),
        in_specs=[
            pl.BlockSpec((R, H), lambda i: (i, 0)),
            pl.BlockSpec((R, 1), lambda i: (i, 0)),
        ],
        out_specs=[
            pl.BlockSpec((R, H), lambda i: (i, 0)),
            pl.BlockSpec((R, 1), lambda i: (i, 0)),
        ],
        out_shape=[
            jax.ShapeDtypeStruct((NPAD, H), jnp.float32),
            jax.ShapeDtypeStruct((NPAD, 1), jnp.float32),
        ],
    )(xw, deg2)


def _mm2_body(acc_ref, y1_ref, dis_ref, b1_ref, w2_ref, y2_ref):
    dis = dis_ref[...]
    s = acc_ref[0] + acc_ref[1] + y1_ref[...]
    h = jnp.maximum(dis * s + b1_ref[...], 0.0)
    y2_ref[...] = jnp.dot(h, w2_ref[...],
                          preferred_element_type=jnp.float32,
                          precision=lax.Precision.HIGHEST) * dis


def _mm2(acc1, y1, dis, b1, W2):
    R = 1024
    return pl.pallas_call(
        _mm2_body,
        grid=(NPAD // R,),
        in_specs=[
            pl.BlockSpec((NC, R, H), lambda i: (0, i, 0)),
            pl.BlockSpec((R, H), lambda i: (i, 0)),
            pl.BlockSpec((R, 1), lambda i: (i, 0)),
            pl.BlockSpec((1, H), lambda i: (0, 0)),
            pl.BlockSpec((H, H), lambda i: (0, 0)),
        ],
        out_specs=pl.BlockSpec((R, H), lambda i: (i, 0)),
        out_shape=jax.ShapeDtypeStruct((NPAD, H), jnp.float32),
    )(acc1, y1, dis, b1, W2)


def _mm3a_body(acc_ref, y2_ref, dis_ref, b2_ref, sw1a_ref, sw1b_ref,
               sb1_ref, z_ref, a_ref, bb_ref):
    dis = dis_ref[...]
    z = dis * (acc_ref[0] + acc_ref[1] + y2_ref[...]) + b2_ref[...]
    z_ref[...] = z
    a_ref[...] = jnp.dot(z, sw1a_ref[...],
                         preferred_element_type=jnp.float32,
                         precision=lax.Precision.HIGHEST) + sb1_ref[...]
    bb_ref[...] = jnp.dot(z, sw1b_ref[...],
                          preferred_element_type=jnp.float32,
                          precision=lax.Precision.HIGHEST)


def _mm3a(acc2, y2, dis, b2, sW1a, sW1b, sb1):
    R = 1024
    return pl.pallas_call(
        _mm3a_body,
        grid=(NPAD // R,),
        in_specs=[
            pl.BlockSpec((NC, R, H), lambda i: (0, i, 0)),
            pl.BlockSpec((R, H), lambda i: (i, 0)),
            pl.BlockSpec((R, 1), lambda i: (i, 0)),
            pl.BlockSpec((1, H), lambda i: (0, 0)),
            pl.BlockSpec((H, H), lambda i: (0, 0)),
            pl.BlockSpec((H, H), lambda i: (0, 0)),
            pl.BlockSpec((1, H), lambda i: (0, 0)),
        ],
        out_specs=[
            pl.BlockSpec((R, H), lambda i: (i, 0)),
            pl.BlockSpec((R, H), lambda i: (i, 0)),
            pl.BlockSpec((R, H), lambda i: (i, 0)),
        ],
        out_shape=[
            jax.ShapeDtypeStruct((NPAD, H), jnp.float32),
            jax.ShapeDtypeStruct((NPAD, H), jnp.float32),
            jax.ShapeDtypeStruct((NPAD, H), jnp.float32),
        ],
    )(acc2, y2, dis, b2, sW1a, sW1b, sb1)


def _mm3b_body(z_ref, fw1_ref, fb1_ref, fw2_ref, fb2_ref, dw1_ref, db1_ref,
               dw2t_ref, db2_ref, fr_ref, pd_ref):
    z = z_ref[...]
    t = jnp.dot(z, fw1_ref[...], preferred_element_type=jnp.float32,
                precision=lax.Precision.HIGHEST) + fb1_ref[...]
    f = jnp.where(t > 0, t, 0.1 * t)
    fr_ref[...] = jnp.dot(f, fw2_ref[...],
                          preferred_element_type=jnp.float32,
                          precision=lax.Precision.HIGHEST) + fb2_ref[...]
    dh = jnp.maximum(jnp.dot(z, dw1_ref[...],
                             preferred_element_type=jnp.float32,
                             precision=lax.Precision.HIGHEST)
                     + db1_ref[...], 0.0)
    pd_ref[...] = jnp.sum(dh * dw2t_ref[...], axis=1, keepdims=True) \
        + db2_ref[...]


def _mm3b(z, fW1, fb1, fW2, fb2, dW1, db1, dW2t, db2):
    R = 1024
    return pl.pallas_call(
        _mm3b_body,
        grid=(NPAD // R,),
        in_specs=[
            pl.BlockSpec((R, H), lambda i: (i, 0)),
            pl.BlockSpec((H, H // 2), lambda i: (0, 0)),
            pl.BlockSpec((1, H // 2), lambda i: (0, 0)),
            pl.BlockSpec((H // 2, D), lambda i: (0, 0)),
            pl.BlockSpec((1, D), lambda i: (0, 0)),
            pl.BlockSpec((H, H), lambda i: (0, 0)),
            pl.BlockSpec((1, H), lambda i: (0, 0)),
            pl.BlockSpec((1, H), lambda i: (0, 0)),
            pl.BlockSpec((1, 1), lambda i: (0, 0)),
        ],
        out_specs=[
            pl.BlockSpec((R, D), lambda i: (i, 0)),
            pl.BlockSpec((R, 1), lambda i: (i, 0)),
        ],
        out_shape=[
            jax.ShapeDtypeStruct((NPAD, D), jnp.float32),
            jax.ShapeDtypeStruct((NPAD, 1), jnp.float32),
        ],
    )(z, fW1, fb1, fW2, fb2, dW1, db1, dW2t, db2)


def kernel(x, edge_index, batch, W1, b1, W2, b2, fW1, fb1, fW2, fb2,
           sW1, sb1, sW2, sb2, dW1, db1, dW2, db2):
    src = edge_index[0]
    dst = edge_index[1]
    pad = jnp.full((EP - E,), N, jnp.int32)
    src3 = jnp.concatenate([src, pad]).reshape(NW, NCHUNK, CH)
    dst3 = jnp.concatenate([dst, pad]).reshape(NW, NCHUNK, CH)
    xp = jnp.pad(x, ((0, NPAD - N), (0, 0)))
    zerosH = jnp.zeros((NPAD, H), jnp.float32)

    xw1 = _mm1(xp, W1)
    degp = _make_deg()(dst3)
    deg2 = (degp[0] + degp[1] + 1.0)[:, None]

    y1, dis = _scale1(xw1, deg2)
    acc1 = _make_prop()(y1, src3, dst3, zerosH)
    y2 = _mm2(acc1, y1, dis, b1.reshape(1, H), W2)
    acc2 = _make_prop()(y2, src3, dst3, zerosH)

    z, A, B = _mm3a(acc2, y2, dis, b2.reshape(1, H),
                    sW1[:H], sW1[H:], sb1.reshape(1, H))
    sb2v = jnp.full((16,), sb2[0], jnp.float32)
    sr = _make_struct()(A, B, src3, dst3, sW2.reshape(H), sb2v)
    fr, pd = _mm3b(z, fW1, fb1.reshape(1, H // 2), fW2, fb2.reshape(1, D),
                   dW1, db1.reshape(1, H), dW2.reshape(1, H),
                   db2.reshape(1, 1))

    sr_flat = sr[:, :EPW].reshape(EP)[:E]
    return (z[:N], fr[:N], sr_flat.reshape(E, 1), pd[:N])


# speedup vs baseline: 2.6179x; 2.6179x over previous
import functools

import jax
import jax.numpy as jnp
from jax import lax
from jax.experimental import pallas as pl
from jax.experimental.pallas import tpu as pltpu
from jax.experimental.pallas import tpu_sc as plsc

N = 10000
E = 320000
D = 128
H = 128
NPAD = 10240
NC = 2
NS = 16
NW = NC * NS
CH = 128
NCHUNK = 80
EPW = NCHUNK * CH
EP = NW * EPW
NBUF = 4
GROUPS = NCHUNK // NBUF
RPW = NPAD // NS


def _mesh():
    return plsc.VectorSubcoreMesh(
        core_axis_name="c", subcore_axis_name="s",
        num_cores=NC, num_subcores=NS)


def _cparams():
    return pltpu.CompilerParams(needs_layout_passes=False)


def _worker():
    cid = lax.axis_index("c")
    sid = lax.axis_index("s")
    return cid, sid, cid * NS + sid


@functools.cache
def _make_deg():
    @functools.partial(
        pl.kernel,
        out_type=jax.ShapeDtypeStruct((NC, NPAD), jnp.float32),
        mesh=_mesh(),
        compiler_params=_cparams(),
        scratch_types=[
            pltpu.VMEM((NCHUNK, CH), jnp.int32),
            pltpu.VMEM((NPAD,), jnp.float32),
            pltpu.VMEM((NS, RPW), jnp.float32),
            pltpu.VMEM((RPW,), jnp.float32),
            pltpu.VMEM_SHARED((NS, NPAD), jnp.float32),
        ],
    )
    def deg_kernel(dst_hbm, out_hbm, idx2, acc_t, red, out_v, part_sh):
        cid, sid, wid = _worker()
        pltpu.sync_copy(dst_hbm.at[wid], idx2)

        @pl.loop(0, NPAD // 16)
        def _zero(i):
            acc_t[pl.ds(i * 16, 16)] = jnp.zeros((16,), jnp.float32)

        ones = jnp.ones((16,), jnp.float32)

        @pl.loop(0, NCHUNK)
        def _chunks(j):
            @pl.loop(0, CH // 16)
            def _vecs(k):
                idxv = idx2[j, pl.ds(k * 16, 16)]
                plsc.addupdate_scatter(acc_t, [idxv], ones)

        pltpu.sync_copy(acc_t, part_sh.at[sid])
        plsc.subcore_barrier()
        for s in range(NS):
            pltpu.sync_copy(part_sh.at[s, pl.ds(sid * RPW, RPW)], red.at[s])

        @pl.loop(0, RPW // 16)
        def _red(i):
            v = jnp.zeros((16,), jnp.float32)
            for s in range(NS):
                v = v + red[s, pl.ds(i * 16, 16)]
            out_v[pl.ds(i * 16, 16)] = v

        pltpu.sync_copy(out_v, out_hbm.at[cid, pl.ds(sid * RPW, RPW)])

    return deg_kernel


@functools.cache
def _make_prop():
    @functools.partial(
        pl.kernel,
        out_type=jax.ShapeDtypeStruct((NC, NPAD, H), jnp.float32),
        mesh=_mesh(),
        compiler_params=_cparams(),
        scratch_types=[
            pltpu.VMEM((2, CH), jnp.int32),
            pltpu.VMEM((2, CH), jnp.int32),
            pltpu.VMEM((2, CH, H), jnp.float32),
            pltpu.VMEM_SHARED((NPAD, H), jnp.float32),
            pltpu.SemaphoreType.DMA((2,)),
            pltpu.SemaphoreType.DMA((2,)),
        ],
    )
    def prop_kernel(y_hbm, src_hbm, dst_hbm, zeros_hbm, out_hbm,
                    idx_s, idx_d, rows2, acc_sh, sem_i, sem_r):
        cid, sid, wid = _worker()
        pltpu.sync_copy(zeros_hbm.at[pl.ds(sid * RPW, RPW), :],
                        acc_sh.at[pl.ds(sid * RPW, RPW), :])
        pltpu.sync_copy(src_hbm.at[wid, 0], idx_s.at[0])
        pltpu.sync_copy(dst_hbm.at[wid, 0], idx_d.at[0])
        plsc.subcore_barrier()
        pltpu.async_copy(y_hbm.at[idx_s.at[0]], rows2.at[0], sem_r.at[0])

        @pl.loop(0, NCHUNK)
        def _chunks(j):
            par = lax.rem(j, 2)
            opar = 1 - par
            nxt = jnp.minimum(j + 1, NCHUNK - 1)
            pltpu.async_copy(src_hbm.at[wid, nxt], idx_s.at[opar],
                             sem_i.at[opar])
            pltpu.async_copy(dst_hbm.at[wid, nxt], idx_d.at[opar],
                             sem_i.at[opar])
            pltpu.make_async_copy(
                y_hbm.at[idx_s.at[0]], rows2.at[par], sem_r.at[par]).wait()
            pltpu.sync_copy(rows2.at[par], acc_sh.at[idx_d.at[par]],
                            add=True)
            pltpu.make_async_copy(src_hbm.at[wid, 0], idx_s.at[opar],
                                  sem_i.at[opar]).wait()
            pltpu.make_async_copy(dst_hbm.at[wid, 0], idx_d.at[opar],
                                  sem_i.at[opar]).wait()
            pltpu.async_copy(y_hbm.at[idx_s.at[opar]], rows2.at[opar],
                             sem_r.at[opar])

        pltpu.make_async_copy(
            y_hbm.at[idx_s.at[0]], rows2.at[0], sem_r.at[0]).wait()

        plsc.subcore_barrier()
        pltpu.sync_copy(acc_sh.at[pl.ds(sid * RPW, RPW), :],
                        out_hbm.at[cid, pl.ds(sid * RPW, RPW), :])

    return prop_kernel


@functools.cache
def _make_struct():
    @functools.partial(
        pl.kernel,
        out_type=jax.ShapeDtypeStruct((NW, (NCHUNK + 1) * CH), jnp.float32),
        mesh=_mesh(),
        compiler_params=_cparams(),
        scratch_types=[
            pltpu.VMEM((NCHUNK, CH), jnp.int32),
            pltpu.VMEM((NCHUNK, CH), jnp.int32),
        ] + [pltpu.VMEM((CH, H), jnp.float32)] * 4
          + [
            pltpu.VMEM((H,), jnp.float32),
            pltpu.VMEM((16,), jnp.float32),
            pltpu.VMEM((CH,), jnp.float32),
            pltpu.VMEM((CH,), jnp.float32),
            pltpu.VMEM((256,), jnp.float32),
        ] + [pltpu.SemaphoreType.DMA] * 6,
    )
    def struct_kernel(a_hbm, b_hbm, src_hbm, dst_hbm, w2_hbm, sb2_hbm,
                      out_hbm, idx_s, idx_d, ra0, rb0, ra1, rb1, w2_v,
                      sb2_v, dv0, dv1, tbuf, ga0, gb0, ga1, gb1, so0, so1):
        rows_a = (ra0, ra1)
        rows_b = (rb0, rb1)
        dot_v = (dv0, dv1)
        sem_a = (ga0, ga1)
        sem_b = (gb0, gb1)
        sem_o = (so0, so1)
        cid, sid, wid = _worker()
        pltpu.sync_copy(w2_hbm, w2_v)
        pltpu.sync_copy(sb2_hbm, sb2_v)
        pltpu.sync_copy(src_hbm.at[wid], idx_s)
        pltpu.sync_copy(dst_hbm.at[wid], idx_d)

        def compute_chunk(b):
            pltpu.make_async_copy(
                a_hbm.at[idx_s.at[0]], rows_a[b], sem_a[b]).wait()
            pltpu.make_async_copy(
                b_hbm.at[idx_d.at[0]], rows_b[b], sem_b[b]).wait()
            pltpu.make_async_copy(
                dot_v[b], out_hbm.at[wid, pl.ds(0, CH)], sem_o[b]).wait()

            @pl.loop(0, CH // 16)
            def _groups(g):
                for e16 in range(16):
                    e = g * 16 + e16
                    acc = jnp.zeros((16,), jnp.float32)
                    for c in range(H // 16):
                        va = rows_a[b][e, pl.ds(c * 16, 16)]
                        vb = rows_b[b][e, pl.ds(c * 16, 16)]
                        t = jnp.maximum(va + vb, 0.0)
                        acc = acc + t * w2_v[pl.ds(c * 16, 16)]
                    tbuf[pl.ds(e16 * 16, 16)] = acc
                flat = lax.iota(jnp.int32, 16) * 16
                vsum = jnp.zeros((16,), jnp.float32)
                for c in range(16):
                    vsum = vsum + plsc.load_gather(tbuf, [flat + c])
                t = vsum + sb2_v[...]
                dot_v[b][pl.ds(g * 16, 16)] = 1.0 / (1.0 + jnp.exp(-t))

        def store_chunk(b, j):
            pltpu.async_copy(dot_v[b], out_hbm.at[wid, pl.ds(j * CH, CH)],
                             sem_o[b])

        def refill(b, j):
            pltpu.async_copy(a_hbm.at[idx_s.at[j]], rows_a[b], sem_a[b])
            pltpu.async_copy(b_hbm.at[idx_d.at[j]], rows_b[b], sem_b[b])

        for b in range(2):
            refill(b, b)
            pltpu.async_copy(dot_v[b],
                             out_hbm.at[wid, pl.ds(NCHUNK * CH, CH)],
                             sem_o[b])

        @pl.loop(0, NCHUNK // 2 - 1)
        def _pairs(p):
            for b in range(2):
                j = p * 2 + b
                compute_chunk(b)
                store_chunk(b, j)
                refill(b, j + 2)

        for b in range(2):
            compute_chunk(b)
            store_chunk(b, NCHUNK - 2 + b)

        for b in range(2):
            pltpu.make_async_copy(
                dot_v[b], out_hbm.at[wid, pl.ds(0, CH)], sem_o[b]).wait()

    return struct_kernel


def _mm1_body(x_ref, w_ref, xw_ref):
    xw_ref[...] = jnp.dot(x_ref[...], w_ref[...],
                          preferred_element_type=jnp.float32,
                          precision=lax.Precision.HIGHEST)


def _mm1(xp, W1):
    R = 1024
    return pl.pallas_call(
        _mm1_body,
        grid=(NPAD // R,),
        in_specs=[
            pl.BlockSpec((R, D), lambda i: (i, 0)),
            pl.BlockSpec((D, H), lambda i: (0, 0)),
        ],
        out_specs=pl.BlockSpec((R, H), lambda i: (i, 0)),
        out_shape=jax.ShapeDtypeStruct((NPAD, H), jnp.float32),
    )(xp, W1)


def _scale_body(xw_ref, deg_ref, y_ref, dis_ref):
    dis = lax.rsqrt(deg_ref[...])
    y_ref[...] = xw_ref[...] * dis
    dis_ref[...] = dis


def _scale1(xw, deg2):
    R = 1024
    return pl.pallas_call(
        _scale_body,
        grid=(NPAD // R,),
        in_specs=[
            pl.BlockSpec((R, H), lambda i: (i, 0)),
            pl.BlockSpec((R, 1), lambda i: (i, 0)),
        ],
        out_specs=[
            pl.BlockSpec((R, H), lambda i: (i, 0)),
            pl.BlockSpec((R, 1), lambda i: (i, 0)),
        ],
        out_shape=[
            jax.ShapeDtypeStruct((NPAD, H), jnp.float32),
            jax.ShapeDtypeStruct((NPAD, 1), jnp.float32),
        ],
    )(xw, deg2)


def _mm2_body(acc_ref, y1_ref, dis_ref, b1_ref, w2_ref, y2_ref):
    dis = dis_ref[...]
    s = acc_ref[0] + acc_ref[1] + y1_ref[...]
    h = jnp.maximum(dis * s + b1_ref[...], 0.0)
    y2_ref[...] = jnp.dot(h, w2_ref[...],
                          preferred_element_type=jnp.float32,
                          precision=lax.Precision.HIGHEST) * dis


def _mm2(acc1, y1, dis, b1, W2):
    R = 1024
    return pl.pallas_call(
        _mm2_body,
        grid=(NPAD // R,),
        in_specs=[
            pl.BlockSpec((NC, R, H), lambda i: (0, i, 0)),
            pl.BlockSpec((R, H), lambda i: (i, 0)),
            pl.BlockSpec((R, 1), lambda i: (i, 0)),
            pl.BlockSpec((1, H), lambda i: (0, 0)),
            pl.BlockSpec((H, H), lambda i: (0, 0)),
        ],
        out_specs=pl.BlockSpec((R, H), lambda i: (i, 0)),
        out_shape=jax.ShapeDtypeStruct((NPAD, H), jnp.float32),
    )(acc1, y1, dis, b1, W2)


def _mm3a_body(acc_ref, y2_ref, dis_ref, b2_ref, sw1a_ref, sw1b_ref,
               sb1_ref, z_ref, a_ref, bb_ref):
    dis = dis_ref[...]
    z = dis * (acc_ref[0] + acc_ref[1] + y2_ref[...]) + b2_ref[...]
    z_ref[...] = z
    a_ref[...] = jnp.dot(z, sw1a_ref[...],
                         preferred_element_type=jnp.float32,
                         precision=lax.Precision.HIGHEST) + sb1_ref[...]
    bb_ref[...] = jnp.dot(z, sw1b_ref[...],
                          preferred_element_type=jnp.float32,
                          precision=lax.Precision.HIGHEST)


def _mm3a(acc2, y2, dis, b2, sW1a, sW1b, sb1):
    R = 1024
    return pl.pallas_call(
        _mm3a_body,
        grid=(NPAD // R,),
        in_specs=[
            pl.BlockSpec((NC, R, H), lambda i: (0, i, 0)),
            pl.BlockSpec((R, H), lambda i: (i, 0)),
            pl.BlockSpec((R, 1), lambda i: (i, 0)),
            pl.BlockSpec((1, H), lambda i: (0, 0)),
            pl.BlockSpec((H, H), lambda i: (0, 0)),
            pl.BlockSpec((H, H), lambda i: (0, 0)),
            pl.BlockSpec((1, H), lambda i: (0, 0)),
        ],
        out_specs=[
            pl.BlockSpec((R, H), lambda i: (i, 0)),
            pl.BlockSpec((R, H), lambda i: (i, 0)),
            pl.BlockSpec((R, H), lambda i: (i, 0)),
        ],
        out_shape=[
            jax.ShapeDtypeStruct((NPAD, H), jnp.float32),
            jax.ShapeDtypeStruct((NPAD, H), jnp.float32),
            jax.ShapeDtypeStruct((NPAD, H), jnp.float32),
        ],
    )(acc2, y2, dis, b2, sW1a, sW1b, sb1)


def _mm3b_body(z_ref, fw1_ref, fb1_ref, fw2_ref, fb2_ref, dw1_ref, db1_ref,
               dw2t_ref, db2_ref, fr_ref, pd_ref):
    z = z_ref[...]
    t = jnp.dot(z, fw1_ref[...], preferred_element_type=jnp.float32,
                precision=lax.Precision.HIGHEST) + fb1_ref[...]
    f = jnp.where(t > 0, t, 0.1 * t)
    fr_ref[...] = jnp.dot(f, fw2_ref[...],
                          preferred_element_type=jnp.float32,
                          precision=lax.Precision.HIGHEST) + fb2_ref[...]
    dh = jnp.maximum(jnp.dot(z, dw1_ref[...],
                             preferred_element_type=jnp.float32,
                             precision=lax.Precision.HIGHEST)
                     + db1_ref[...], 0.0)
    pd_ref[...] = jnp.sum(dh * dw2t_ref[...], axis=1, keepdims=True) \
        + db2_ref[...]


def _mm3b(z, fW1, fb1, fW2, fb2, dW1, db1, dW2t, db2):
    R = 1024
    return pl.pallas_call(
        _mm3b_body,
        grid=(NPAD // R,),
        in_specs=[
            pl.BlockSpec((R, H), lambda i: (i, 0)),
            pl.BlockSpec((H, H // 2), lambda i: (0, 0)),
            pl.BlockSpec((1, H // 2), lambda i: (0, 0)),
            pl.BlockSpec((H // 2, D), lambda i: (0, 0)),
            pl.BlockSpec((1, D), lambda i: (0, 0)),
            pl.BlockSpec((H, H), lambda i: (0, 0)),
            pl.BlockSpec((1, H), lambda i: (0, 0)),
            pl.BlockSpec((1, H), lambda i: (0, 0)),
            pl.BlockSpec((1, 1), lambda i: (0, 0)),
        ],
        out_specs=[
            pl.BlockSpec((R, D), lambda i: (i, 0)),
            pl.BlockSpec((R, 1), lambda i: (i, 0)),
        ],
        out_shape=[
            jax.ShapeDtypeStruct((NPAD, D), jnp.float32),
            jax.ShapeDtypeStruct((NPAD, 1), jnp.float32),
        ],
    )(z, fW1, fb1, fW2, fb2, dW1, db1, dW2t, db2)


def kernel(x, edge_index, batch, W1, b1, W2, b2, fW1, fb1, fW2, fb2,
           sW1, sb1, sW2, sb2, dW1, db1, dW2, db2):
    src = edge_index[0]
    dst = edge_index[1]
    pad = N + (jnp.arange(EP - E, dtype=jnp.int32) % (NPAD - N))
    src3 = jnp.concatenate([src, pad]).reshape(NW, NCHUNK, CH)
    dst3 = jnp.concatenate([dst, pad]).reshape(NW, NCHUNK, CH)
    xp = jnp.pad(x, ((0, NPAD - N), (0, 0)))
    zerosH = jnp.zeros((NPAD, H), jnp.float32)

    xw1 = _mm1(xp, W1)
    degp = _make_deg()(dst3)
    deg2 = (degp[0] + degp[1] + 1.0)[:, None]

    y1, dis = _scale1(xw1, deg2)
    acc1 = _make_prop()(y1, src3, dst3, zerosH)
    y2 = _mm2(acc1, y1, dis, b1.reshape(1, H), W2)
    acc2 = _make_prop()(y2, src3, dst3, zerosH)

    z, A, B = _mm3a(acc2, y2, dis, b2.reshape(1, H),
                    sW1[:H], sW1[H:], sb1.reshape(1, H))
    sb2v = jnp.full((16,), sb2[0], jnp.float32)
    sr = _make_struct()(A, B, src3, dst3, sW2.reshape(H), sb2v)
    fr, pd = _mm3b(z, fW1, fb1.reshape(1, H // 2), fW2, fb2.reshape(1, D),
                   dW1, db1.reshape(1, H), dW2.reshape(1, H),
                   db2.reshape(1, 1))

    sr_flat = sr[:, :EPW].reshape(EP)[:E]
    return (z[:N], fr[:N], sr_flat.reshape(E, 1), pd[:N])


# speedup vs baseline: 3.2075x; 1.2252x over previous
import functools

import jax
import jax.numpy as jnp
from jax import lax
from jax.experimental import pallas as pl
from jax.experimental.pallas import tpu as pltpu
from jax.experimental.pallas import tpu_sc as plsc

N = 10000
E = 320000
D = 128
H = 128
NPAD = 10240
NC = 2
NS = 16
NW = NC * NS
CH = 128
NCHUNK = 80
EPW = NCHUNK * CH
EP = NW * EPW
PBUF = 4
PCH = 64
PNCHUNK = EPW // PCH
RPW = NPAD // NS


def _mesh():
    return plsc.VectorSubcoreMesh(
        core_axis_name="c", subcore_axis_name="s",
        num_cores=NC, num_subcores=NS)


def _cparams():
    return pltpu.CompilerParams(needs_layout_passes=False)


def _worker():
    cid = lax.axis_index("c")
    sid = lax.axis_index("s")
    return cid, sid, cid * NS + sid


@functools.cache
def _make_deg():
    @functools.partial(
        pl.kernel,
        out_type=jax.ShapeDtypeStruct((NC, NPAD), jnp.float32),
        mesh=_mesh(),
        compiler_params=_cparams(),
        scratch_types=[
            pltpu.VMEM((NCHUNK, CH), jnp.int32),
            pltpu.VMEM((NPAD,), jnp.float32),
            pltpu.VMEM((NS, RPW), jnp.float32),
            pltpu.VMEM((RPW,), jnp.float32),
            pltpu.VMEM_SHARED((NS, NPAD), jnp.float32),
        ],
    )
    def deg_kernel(dst_hbm, out_hbm, idx2, acc_t, red, out_v, part_sh):
        cid, sid, wid = _worker()
        pltpu.sync_copy(dst_hbm.at[wid], idx2)

        @pl.loop(0, NPAD // 16)
        def _zero(i):
            acc_t[pl.ds(i * 16, 16)] = jnp.zeros((16,), jnp.float32)

        ones = jnp.ones((16,), jnp.float32)

        @pl.loop(0, NCHUNK)
        def _chunks(j):
            @pl.loop(0, CH // 16)
            def _vecs(k):
                idxv = idx2[j, pl.ds(k * 16, 16)]
                plsc.addupdate_scatter(acc_t, [idxv], ones)

        pltpu.sync_copy(acc_t, part_sh.at[sid])
        plsc.subcore_barrier()
        for s in range(NS):
            pltpu.sync_copy(part_sh.at[s, pl.ds(sid * RPW, RPW)], red.at[s])

        @pl.loop(0, RPW // 16)
        def _red(i):
            v = jnp.zeros((16,), jnp.float32)
            for s in range(NS):
                v = v + red[s, pl.ds(i * 16, 16)]
            out_v[pl.ds(i * 16, 16)] = v

        pltpu.sync_copy(out_v, out_hbm.at[cid, pl.ds(sid * RPW, RPW)])

    return deg_kernel


@functools.cache
def _make_prop():
    @functools.partial(
        pl.kernel,
        out_type=jax.ShapeDtypeStruct((NC, NPAD, H), jnp.float32),
        mesh=_mesh(),
        compiler_params=_cparams(),
        scratch_types=[
            pltpu.VMEM((PBUF, PCH), jnp.int32),
            pltpu.VMEM((PBUF, PCH), jnp.int32),
            pltpu.VMEM((PBUF, PCH, H), jnp.float32),
            pltpu.VMEM_SHARED((NPAD, H), jnp.float32),
            pltpu.SemaphoreType.DMA((PBUF,)),
            pltpu.SemaphoreType.DMA((PBUF,)),
        ],
    )
    def prop_kernel(y_hbm, src_hbm, dst_hbm, zeros_hbm, out_hbm,
                    idx_s, idx_d, rows2, acc_sh, sem_i, sem_r):
        cid, sid, wid = _worker()
        pltpu.sync_copy(zeros_hbm.at[pl.ds(sid * RPW, RPW), :],
                        acc_sh.at[pl.ds(sid * RPW, RPW), :])
        plsc.subcore_barrier()
        for b in range(PBUF - 1):
            pltpu.sync_copy(src_hbm.at[wid, b], idx_s.at[b])
            pltpu.sync_copy(dst_hbm.at[wid, b], idx_d.at[b])
            pltpu.async_copy(y_hbm.at[idx_s.at[b]], rows2.at[b],
                             sem_r.at[b])

        @pl.loop(0, PNCHUNK)
        def _chunks(j):
            par = lax.rem(j, PBUF)
            fpar = lax.rem(j + PBUF - 1, PBUF)
            nxt = jnp.minimum(j + PBUF - 1, PNCHUNK - 1)
            pltpu.async_copy(src_hbm.at[wid, nxt], idx_s.at[fpar],
                             sem_i.at[fpar])
            pltpu.async_copy(dst_hbm.at[wid, nxt], idx_d.at[fpar],
                             sem_i.at[fpar])
            pltpu.make_async_copy(
                y_hbm.at[idx_s.at[0]], rows2.at[par], sem_r.at[par]).wait()
            pltpu.sync_copy(rows2.at[par], acc_sh.at[idx_d.at[par]],
                            add=True)
            pltpu.make_async_copy(src_hbm.at[wid, 0], idx_s.at[fpar],
                                  sem_i.at[fpar]).wait()
            pltpu.make_async_copy(dst_hbm.at[wid, 0], idx_d.at[fpar],
                                  sem_i.at[fpar]).wait()
            pltpu.async_copy(y_hbm.at[idx_s.at[fpar]], rows2.at[fpar],
                             sem_r.at[fpar])

        @pl.loop(0, PBUF - 1)
        def _drain(k):
            par = lax.rem(PNCHUNK + k, PBUF)
            pltpu.make_async_copy(
                y_hbm.at[idx_s.at[0]], rows2.at[par], sem_r.at[par]).wait()

        plsc.subcore_barrier()
        pltpu.sync_copy(acc_sh.at[pl.ds(sid * RPW, RPW), :],
                        out_hbm.at[cid, pl.ds(sid * RPW, RPW), :])

    return prop_kernel


@functools.cache
def _make_struct():
    @functools.partial(
        pl.kernel,
        out_type=jax.ShapeDtypeStruct((NW, (NCHUNK + 1) * CH), jnp.float32),
        mesh=_mesh(),
        compiler_params=_cparams(),
        scratch_types=[
            pltpu.VMEM((NCHUNK, CH), jnp.int32),
            pltpu.VMEM((NCHUNK, CH), jnp.int32),
        ] + [pltpu.VMEM((CH, H), jnp.float32)] * 4
          + [
            pltpu.VMEM((H,), jnp.float32),
            pltpu.VMEM((16,), jnp.float32),
            pltpu.VMEM((CH,), jnp.float32),
            pltpu.VMEM((CH,), jnp.float32),
            pltpu.VMEM((256,), jnp.float32),
        ] + [pltpu.SemaphoreType.DMA] * 6,
    )
    def struct_kernel(a_hbm, b_hbm, src_hbm, dst_hbm, w2_hbm, sb2_hbm,
                      out_hbm, idx_s, idx_d, ra0, rb0, ra1, rb1, w2_v,
                      sb2_v, dv0, dv1, tbuf, ga0, gb0, ga1, gb1, so0, so1):
        rows_a = (ra0, ra1)
        rows_b = (rb0, rb1)
        dot_v = (dv0, dv1)
        sem_a = (ga0, ga1)
        sem_b = (gb0, gb1)
        sem_o = (so0, so1)
        cid, sid, wid = _worker()
        pltpu.sync_copy(w2_hbm, w2_v)
        pltpu.sync_copy(sb2_hbm, sb2_v)
        pltpu.sync_copy(src_hbm.at[wid], idx_s)
        pltpu.sync_copy(dst_hbm.at[wid], idx_d)

        def compute_chunk(b):
            pltpu.make_async_copy(
                a_hbm.at[idx_s.at[0]], rows_a[b], sem_a[b]).wait()
            pltpu.make_async_copy(
                b_hbm.at[idx_d.at[0]], rows_b[b], sem_b[b]).wait()
            pltpu.make_async_copy(
                dot_v[b], out_hbm.at[wid, pl.ds(0, CH)], sem_o[b]).wait()

            @pl.loop(0, CH // 16)
            def _groups(g):
                for e16 in range(16):
                    e = g * 16 + e16
                    acc = jnp.zeros((16,), jnp.float32)
                    for c in range(H // 16):
                        va = rows_a[b][e, pl.ds(c * 16, 16)]
                        vb = rows_b[b][e, pl.ds(c * 16, 16)]
                        t = jnp.maximum(va + vb, 0.0)
                        acc = acc + t * w2_v[pl.ds(c * 16, 16)]
                    tbuf[pl.ds(e16 * 16, 16)] = acc
                flat = lax.iota(jnp.int32, 16) * 16
                vsum = jnp.zeros((16,), jnp.float32)
                for c in range(16):
                    vsum = vsum + plsc.load_gather(tbuf, [flat + c])
                t = vsum + sb2_v[...]
                dot_v[b][pl.ds(g * 16, 16)] = 1.0 / (1.0 + jnp.exp(-t))

        def store_chunk(b, j):
            pltpu.async_copy(dot_v[b], out_hbm.at[wid, pl.ds(j * CH, CH)],
                             sem_o[b])

        def refill(b, j):
            pltpu.async_copy(a_hbm.at[idx_s.at[j]], rows_a[b], sem_a[b])
            pltpu.async_copy(b_hbm.at[idx_d.at[j]], rows_b[b], sem_b[b])

        for b in range(2):
            refill(b, b)
            pltpu.async_copy(dot_v[b],
                             out_hbm.at[wid, pl.ds(NCHUNK * CH, CH)],
                             sem_o[b])

        @pl.loop(0, NCHUNK // 2 - 1)
        def _pairs(p):
            for b in range(2):
                j = p * 2 + b
                compute_chunk(b)
                store_chunk(b, j)
                refill(b, j + 2)

        for b in range(2):
            compute_chunk(b)
            store_chunk(b, NCHUNK - 2 + b)

        for b in range(2):
            pltpu.make_async_copy(
                dot_v[b], out_hbm.at[wid, pl.ds(0, CH)], sem_o[b]).wait()

    return struct_kernel


def _mm1_body(x_ref, w_ref, xw_ref):
    xw_ref[...] = jnp.dot(x_ref[...], w_ref[...],
                          preferred_element_type=jnp.float32,
                          precision=lax.Precision.HIGHEST)


def _mm1(xp, W1):
    R = 1024
    return pl.pallas_call(
        _mm1_body,
        grid=(NPAD // R,),
        in_specs=[
            pl.BlockSpec((R, D), lambda i: (i, 0)),
            pl.BlockSpec((D, H), lambda i: (0, 0)),
        ],
        out_specs=pl.BlockSpec((R, H), lambda i: (i, 0)),
        out_shape=jax.ShapeDtypeStruct((NPAD, H), jnp.float32),
    )(xp, W1)


def _scale_body(xw_ref, deg_ref, y_ref, dis_ref):
    dis = lax.rsqrt(deg_ref[...])
    y_ref[...] = xw_ref[...] * dis
    dis_ref[...] = dis


def _scale1(xw, deg2):
    R = 1024
    return pl.pallas_call(
        _scale_body,
        grid=(NPAD // R,),
        in_specs=[
            pl.BlockSpec((R, H), lambda i: (i, 0)),
            pl.BlockSpec((R, 1), lambda i: (i, 0)),
        ],
        out_specs=[
            pl.BlockSpec((R, H), lambda i: (i, 0)),
            pl.BlockSpec((R, 1), lambda i: (i, 0)),
        ],
        out_shape=[
            jax.ShapeDtypeStruct((NPAD, H), jnp.float32),
            jax.ShapeDtypeStruct((NPAD, 1), jnp.float32),
        ],
    )(xw, deg2)


def _mm2_body(acc_ref, y1_ref, dis_ref, b1_ref, w2_ref, y2_ref):
    dis = dis_ref[...]
    s = acc_ref[0] + acc_ref[1] + y1_ref[...]
    h = jnp.maximum(dis * s + b1_ref[...], 0.0)
    y2_ref[...] = jnp.dot(h, w2_ref[...],
                          preferred_element_type=jnp.float32,
                          precision=lax.Precision.HIGHEST) * dis


def _mm2(acc1, y1, dis, b1, W2):
    R = 1024
    return pl.pallas_call(
        _mm2_body,
        grid=(NPAD // R,),
        in_specs=[
            pl.BlockSpec((NC, R, H), lambda i: (0, i, 0)),
            pl.BlockSpec((R, H), lambda i: (i, 0)),
            pl.BlockSpec((R, 1), lambda i: (i, 0)),
            pl.BlockSpec((1, H), lambda i: (0, 0)),
            pl.BlockSpec((H, H), lambda i: (0, 0)),
        ],
        out_specs=pl.BlockSpec((R, H), lambda i: (i, 0)),
        out_shape=jax.ShapeDtypeStruct((NPAD, H), jnp.float32),
    )(acc1, y1, dis, b1, W2)


def _mm3a_body(acc_ref, y2_ref, dis_ref, b2_ref, sw1a_ref, sw1b_ref,
               sb1_ref, z_ref, a_ref, bb_ref):
    dis = dis_ref[...]
    z = dis * (acc_ref[0] + acc_ref[1] + y2_ref[...]) + b2_ref[...]
    z_ref[...] = z
    a_ref[...] = jnp.dot(z, sw1a_ref[...],
                         preferred_element_type=jnp.float32,
                         precision=lax.Precision.HIGHEST) + sb1_ref[...]
    bb_ref[...] = jnp.dot(z, sw1b_ref[...],
                          preferred_element_type=jnp.float32,
                          precision=lax.Precision.HIGHEST)


def _mm3a(acc2, y2, dis, b2, sW1a, sW1b, sb1):
    R = 1024
    return pl.pallas_call(
        _mm3a_body,
        grid=(NPAD // R,),
        in_specs=[
            pl.BlockSpec((NC, R, H), lambda i: (0, i, 0)),
            pl.BlockSpec((R, H), lambda i: (i, 0)),
            pl.BlockSpec((R, 1), lambda i: (i, 0)),
            pl.BlockSpec((1, H), lambda i: (0, 0)),
            pl.BlockSpec((H, H), lambda i: (0, 0)),
            pl.BlockSpec((H, H), lambda i: (0, 0)),
            pl.BlockSpec((1, H), lambda i: (0, 0)),
        ],
        out_specs=[
            pl.BlockSpec((R, H), lambda i: (i, 0)),
            pl.BlockSpec((R, H), lambda i: (i, 0)),
            pl.BlockSpec((R, H), lambda i: (i, 0)),
        ],
        out_shape=[
            jax.ShapeDtypeStruct((NPAD, H), jnp.float32),
            jax.ShapeDtypeStruct((NPAD, H), jnp.float32),
            jax.ShapeDtypeStruct((NPAD, H), jnp.float32),
        ],
    )(acc2, y2, dis, b2, sW1a, sW1b, sb1)


def _mm3b_body(z_ref, fw1_ref, fb1_ref, fw2_ref, fb2_ref, dw1_ref, db1_ref,
               dw2t_ref, db2_ref, fr_ref, pd_ref):
    z = z_ref[...]
    t = jnp.dot(z, fw1_ref[...], preferred_element_type=jnp.float32,
                precision=lax.Precision.HIGHEST) + fb1_ref[...]
    f = jnp.where(t > 0, t, 0.1 * t)
    fr_ref[...] = jnp.dot(f, fw2_ref[...],
                          preferred_element_type=jnp.float32,
                          precision=lax.Precision.HIGHEST) + fb2_ref[...]
    dh = jnp.maximum(jnp.dot(z, dw1_ref[...],
                             preferred_element_type=jnp.float32,
                             precision=lax.Precision.HIGHEST)
                     + db1_ref[...], 0.0)
    pd_ref[...] = jnp.sum(dh * dw2t_ref[...], axis=1, keepdims=True) \
        + db2_ref[...]


def _mm3b(z, fW1, fb1, fW2, fb2, dW1, db1, dW2t, db2):
    R = 1024
    return pl.pallas_call(
        _mm3b_body,
        grid=(NPAD // R,),
        in_specs=[
            pl.BlockSpec((R, H), lambda i: (i, 0)),
            pl.BlockSpec((H, H // 2), lambda i: (0, 0)),
            pl.BlockSpec((1, H // 2), lambda i: (0, 0)),
            pl.BlockSpec((H // 2, D), lambda i: (0, 0)),
            pl.BlockSpec((1, D), lambda i: (0, 0)),
            pl.BlockSpec((H, H), lambda i: (0, 0)),
            pl.BlockSpec((1, H), lambda i: (0, 0)),
            pl.BlockSpec((1, H), lambda i: (0, 0)),
            pl.BlockSpec((1, 1), lambda i: (0, 0)),
        ],
        out_specs=[
            pl.BlockSpec((R, D), lambda i: (i, 0)),
            pl.BlockSpec((R, 1), lambda i: (i, 0)),
        ],
        out_shape=[
            jax.ShapeDtypeStruct((NPAD, D), jnp.float32),
            jax.ShapeDtypeStruct((NPAD, 1), jnp.float32),
        ],
    )(z, fW1, fb1, fW2, fb2, dW1, db1, dW2t, db2)


def kernel(x, edge_index, batch, W1, b1, W2, b2, fW1, fb1, fW2, fb2,
           sW1, sb1, sW2, sb2, dW1, db1, dW2, db2):
    src = edge_index[0]
    dst = edge_index[1]
    pad = N + (jnp.arange(EP - E, dtype=jnp.int32) % (NPAD - N))
    src3 = jnp.concatenate([src, pad]).reshape(NW, NCHUNK, CH)
    dst3 = jnp.concatenate([dst, pad]).reshape(NW, NCHUNK, CH)
    xp = jnp.pad(x, ((0, NPAD - N), (0, 0)))
    zerosH = jnp.zeros((NPAD, H), jnp.float32)

    xw1 = _mm1(xp, W1)
    degp = _make_deg()(dst3)
    deg2 = (degp[0] + degp[1] + 1.0)[:, None]

    srcp = src3.reshape(NW, PNCHUNK, PCH)
    dstp = dst3.reshape(NW, PNCHUNK, PCH)
    y1, dis = _scale1(xw1, deg2)
    acc1 = _make_prop()(y1, srcp, dstp, zerosH)
    y2 = _mm2(acc1, y1, dis, b1.reshape(1, H), W2)
    acc2 = _make_prop()(y2, srcp, dstp, zerosH)

    z, A, B = _mm3a(acc2, y2, dis, b2.reshape(1, H),
                    sW1[:H], sW1[H:], sb1.reshape(1, H))
    sb2v = jnp.full((16,), sb2[0], jnp.float32)
    sr = _make_struct()(A, B, src3, dst3, sW2.reshape(H), sb2v)
    fr, pd = _mm3b(z, fW1, fb1.reshape(1, H // 2), fW2, fb2.reshape(1, D),
                   dW1, db1.reshape(1, H), dW2.reshape(1, H),
                   db2.reshape(1, 1))

    sr_flat = sr[:, :EPW].reshape(EP)[:E]
    return (z[:N], fr[:N], sr_flat.reshape(E, 1), pd[:N])


# speedup vs baseline: 3.2554x; 1.0149x over previous
import functools

import jax
import jax.numpy as jnp
from jax import lax
from jax.experimental import pallas as pl
from jax.experimental.pallas import tpu as pltpu
from jax.experimental.pallas import tpu_sc as plsc

N = 10000
E = 320000
D = 128
H = 128
NPAD = 10240
NC = 2
NS = 16
NW = NC * NS
CH = 128
NCHUNK = 80
EPW = NCHUNK * CH
EP = NW * EPW
PBUF = 4
PCH = 64
PNCHUNK = EPW // PCH
RPW = NPAD // NS


def _mesh():
    return plsc.VectorSubcoreMesh(
        core_axis_name="c", subcore_axis_name="s",
        num_cores=NC, num_subcores=NS)


def _cparams():
    return pltpu.CompilerParams(needs_layout_passes=False)


def _worker():
    cid = lax.axis_index("c")
    sid = lax.axis_index("s")
    return cid, sid, cid * NS + sid


@functools.cache
def _make_deg():
    @functools.partial(
        pl.kernel,
        out_type=jax.ShapeDtypeStruct((NC, NPAD), jnp.float32),
        mesh=_mesh(),
        compiler_params=_cparams(),
        scratch_types=[
            pltpu.VMEM((NCHUNK, CH), jnp.int32),
            pltpu.VMEM((NPAD,), jnp.float32),
            pltpu.VMEM((NS, RPW), jnp.float32),
            pltpu.VMEM((RPW,), jnp.float32),
            pltpu.VMEM_SHARED((NS, NPAD), jnp.float32),
        ],
    )
    def deg_kernel(dst_hbm, out_hbm, idx2, acc_t, red, out_v, part_sh):
        cid, sid, wid = _worker()
        pltpu.sync_copy(dst_hbm.at[wid], idx2)

        @pl.loop(0, NPAD // 16)
        def _zero(i):
            acc_t[pl.ds(i * 16, 16)] = jnp.zeros((16,), jnp.float32)

        ones = jnp.ones((16,), jnp.float32)

        @pl.loop(0, NCHUNK)
        def _chunks(j):
            @pl.loop(0, CH // 16)
            def _vecs(k):
                idxv = idx2[j, pl.ds(k * 16, 16)]
                plsc.addupdate_scatter(acc_t, [idxv], ones)

        pltpu.sync_copy(acc_t, part_sh.at[sid])
        plsc.subcore_barrier()
        for s in range(NS):
            pltpu.sync_copy(part_sh.at[s, pl.ds(sid * RPW, RPW)], red.at[s])

        @pl.loop(0, RPW // 16)
        def _red(i):
            v = jnp.zeros((16,), jnp.float32)
            for s in range(NS):
                v = v + red[s, pl.ds(i * 16, 16)]
            out_v[pl.ds(i * 16, 16)] = v

        pltpu.sync_copy(out_v, out_hbm.at[cid, pl.ds(sid * RPW, RPW)])

    return deg_kernel


@functools.cache
def _make_prop():
    @functools.partial(
        pl.kernel,
        out_type=jax.ShapeDtypeStruct((NC, NPAD, H), jnp.float32),
        mesh=_mesh(),
        compiler_params=_cparams(),
        scratch_types=[
            pltpu.VMEM((PBUF, PCH), jnp.int32),
            pltpu.VMEM((PBUF, PCH), jnp.int32),
            pltpu.VMEM((PBUF, PCH, H), jnp.float32),
            pltpu.VMEM_SHARED((NPAD, H), jnp.float32),
            pltpu.SemaphoreType.DMA((PBUF,)),
            pltpu.SemaphoreType.DMA((PBUF,)),
        ],
    )
    def prop_kernel(y_hbm, src_hbm, dst_hbm, zeros_hbm, out_hbm,
                    idx_s, idx_d, rows2, acc_sh, sem_i, sem_r):
        cid, sid, wid = _worker()
        pltpu.sync_copy(zeros_hbm.at[pl.ds(sid * RPW, RPW), :],
                        acc_sh.at[pl.ds(sid * RPW, RPW), :])
        plsc.subcore_barrier()
        for b in range(PBUF - 1):
            pltpu.sync_copy(src_hbm.at[wid, b], idx_s.at[b])
            pltpu.sync_copy(dst_hbm.at[wid, b], idx_d.at[b])
            pltpu.async_copy(y_hbm.at[idx_s.at[b]], rows2.at[b],
                             sem_r.at[b])

        @pl.loop(0, PNCHUNK)
        def _chunks(j):
            par = lax.rem(j, PBUF)
            fpar = lax.rem(j + PBUF - 1, PBUF)
            nxt = jnp.minimum(j + PBUF - 1, PNCHUNK - 1)
            pltpu.async_copy(src_hbm.at[wid, nxt], idx_s.at[fpar],
                             sem_i.at[fpar])
            pltpu.async_copy(dst_hbm.at[wid, nxt], idx_d.at[fpar],
                             sem_i.at[fpar])
            pltpu.make_async_copy(
                y_hbm.at[idx_s.at[0]], rows2.at[par], sem_r.at[par]).wait()
            pltpu.sync_copy(rows2.at[par], acc_sh.at[idx_d.at[par]],
                            add=True)
            pltpu.make_async_copy(src_hbm.at[wid, 0], idx_s.at[fpar],
                                  sem_i.at[fpar]).wait()
            pltpu.make_async_copy(dst_hbm.at[wid, 0], idx_d.at[fpar],
                                  sem_i.at[fpar]).wait()
            pltpu.async_copy(y_hbm.at[idx_s.at[fpar]], rows2.at[fpar],
                             sem_r.at[fpar])

        @pl.loop(0, PBUF - 1)
        def _drain(k):
            par = lax.rem(PNCHUNK + k, PBUF)
            pltpu.make_async_copy(
                y_hbm.at[idx_s.at[0]], rows2.at[par], sem_r.at[par]).wait()

        plsc.subcore_barrier()
        pltpu.sync_copy(acc_sh.at[pl.ds(sid * RPW, RPW), :],
                        out_hbm.at[cid, pl.ds(sid * RPW, RPW), :])

    return prop_kernel


SBUF = 3


@functools.cache
def _make_struct():
    @functools.partial(
        pl.kernel,
        out_type=jax.ShapeDtypeStruct((NW, (NCHUNK + 1) * CH), jnp.float32),
        mesh=_mesh(),
        compiler_params=_cparams(),
        scratch_types=[
            pltpu.VMEM((NCHUNK, CH), jnp.int32),
            pltpu.VMEM((NCHUNK, CH), jnp.int32),
            pltpu.VMEM((SBUF, CH, H), jnp.float32),
            pltpu.VMEM((SBUF, CH, H), jnp.float32),
            pltpu.VMEM((H,), jnp.float32),
            pltpu.VMEM((16,), jnp.float32),
            pltpu.VMEM((SBUF, CH), jnp.float32),
            pltpu.VMEM((256,), jnp.float32),
            pltpu.SemaphoreType.DMA((SBUF,)),
            pltpu.SemaphoreType.DMA((SBUF,)),
            pltpu.SemaphoreType.DMA((SBUF,)),
        ],
    )
    def struct_kernel(a_hbm, b_hbm, src_hbm, dst_hbm, w2_hbm, sb2_hbm,
                      out_hbm, idx_s, idx_d, ra, rb, w2_v, sb2_v, dv,
                      tbuf, sem_a, sem_b, sem_o):
        cid, sid, wid = _worker()
        pltpu.sync_copy(w2_hbm, w2_v)
        pltpu.sync_copy(sb2_hbm, sb2_v)
        pltpu.sync_copy(src_hbm.at[wid], idx_s)
        pltpu.sync_copy(dst_hbm.at[wid], idx_d)

        for b in range(SBUF - 1):
            pltpu.async_copy(a_hbm.at[idx_s.at[b]], ra.at[b], sem_a.at[b])
            pltpu.async_copy(b_hbm.at[idx_d.at[b]], rb.at[b], sem_b.at[b])
        for b in range(SBUF):
            pltpu.async_copy(dv.at[b],
                             out_hbm.at[wid, pl.ds(NCHUNK * CH, CH)],
                             sem_o.at[b])

        @pl.loop(0, NCHUNK)
        def _chunks(j):
            par = lax.rem(j, SBUF)
            fpar = lax.rem(j + SBUF - 1, SBUF)
            nxt = jnp.minimum(j + SBUF - 1, NCHUNK - 1)
            pltpu.make_async_copy(
                a_hbm.at[idx_s.at[0]], ra.at[par], sem_a.at[par]).wait()
            pltpu.make_async_copy(
                b_hbm.at[idx_d.at[0]], rb.at[par], sem_b.at[par]).wait()
            pltpu.make_async_copy(
                dv.at[par], out_hbm.at[wid, pl.ds(0, CH)],
                sem_o.at[par]).wait()

            @pl.loop(0, CH // 16)
            def _groups(g):
                for e16 in range(16):
                    e = g * 16 + e16
                    acc = jnp.zeros((16,), jnp.float32)
                    for c in range(H // 16):
                        va = ra[par, e, pl.ds(c * 16, 16)]
                        vb = rb[par, e, pl.ds(c * 16, 16)]
                        t = jnp.maximum(va + vb, 0.0)
                        acc = acc + t * w2_v[pl.ds(c * 16, 16)]
                    tbuf[pl.ds(e16 * 16, 16)] = acc
                flat = lax.iota(jnp.int32, 16) * 16
                vsum = jnp.zeros((16,), jnp.float32)
                for c in range(16):
                    vsum = vsum + plsc.load_gather(tbuf, [flat + c])
                t = vsum + sb2_v[...]
                dv[par, pl.ds(g * 16, 16)] = 1.0 / (1.0 + jnp.exp(-t))

            pltpu.async_copy(dv.at[par],
                             out_hbm.at[wid, pl.ds(j * CH, CH)],
                             sem_o.at[par])
            pltpu.async_copy(a_hbm.at[idx_s.at[nxt]], ra.at[fpar],
                             sem_a.at[fpar])
            pltpu.async_copy(b_hbm.at[idx_d.at[nxt]], rb.at[fpar],
                             sem_b.at[fpar])

        @pl.loop(0, SBUF - 1)
        def _draing(k):
            par = lax.rem(NCHUNK + k, SBUF)
            pltpu.make_async_copy(
                a_hbm.at[idx_s.at[0]], ra.at[par], sem_a.at[par]).wait()
            pltpu.make_async_copy(
                b_hbm.at[idx_d.at[0]], rb.at[par], sem_b.at[par]).wait()

        @pl.loop(0, SBUF)
        def _draino(k):
            pltpu.make_async_copy(
                dv.at[lax.rem(k, SBUF)], out_hbm.at[wid, pl.ds(0, CH)],
                sem_o.at[lax.rem(k, SBUF)]).wait()


    return struct_kernel


_PREC = dict(preferred_element_type=jnp.float32)


def _mm1_body(x_ref, w_ref, deg_ref, y_ref, dis_ref):
    dis = lax.rsqrt(deg_ref[...])
    y_ref[...] = jnp.dot(x_ref[...], w_ref[...], **_PREC) * dis
    dis_ref[...] = dis


def _mm1(xp, W1, deg2):
    R = 1024
    return pl.pallas_call(
        _mm1_body,
        grid=(NPAD // R,),
        in_specs=[
            pl.BlockSpec((R, D), lambda i: (i, 0)),
            pl.BlockSpec((D, H), lambda i: (0, 0)),
            pl.BlockSpec((R, 1), lambda i: (i, 0)),
        ],
        out_specs=[
            pl.BlockSpec((R, H), lambda i: (i, 0)),
            pl.BlockSpec((R, 1), lambda i: (i, 0)),
        ],
        out_shape=[
            jax.ShapeDtypeStruct((NPAD, H), jnp.float32),
            jax.ShapeDtypeStruct((NPAD, 1), jnp.float32),
        ],
    )(xp, W1, deg2)


def _mm2_body(acc_ref, y1_ref, dis_ref, b1_ref, w2_ref, y2_ref):
    dis = dis_ref[...]
    s = acc_ref[0] + acc_ref[1] + y1_ref[...]
    h = jnp.maximum(dis * s + b1_ref[...], 0.0)
    y2_ref[...] = jnp.dot(h, w2_ref[...], **_PREC) * dis


def _mm2(acc1, y1, dis, b1, W2):
    R = 1024
    return pl.pallas_call(
        _mm2_body,
        grid=(NPAD // R,),
        in_specs=[
            pl.BlockSpec((NC, R, H), lambda i: (0, i, 0)),
            pl.BlockSpec((R, H), lambda i: (i, 0)),
            pl.BlockSpec((R, 1), lambda i: (i, 0)),
            pl.BlockSpec((1, H), lambda i: (0, 0)),
            pl.BlockSpec((H, H), lambda i: (0, 0)),
        ],
        out_specs=pl.BlockSpec((R, H), lambda i: (i, 0)),
        out_shape=jax.ShapeDtypeStruct((NPAD, H), jnp.float32),
    )(acc1, y1, dis, b1, W2)


def _mm3_body(acc_ref, y2_ref, dis_ref, b2_ref, sw1a_ref, sw1b_ref,
              sb1_ref, fw1_ref, fb1_ref, fw2_ref, fb2_ref, dw1_ref,
              db1_ref, dw2t_ref, db2_ref, z_ref, a_ref, bb_ref, fr_ref,
              pd_ref):
    dis = dis_ref[...]
    z = dis * (acc_ref[0] + acc_ref[1] + y2_ref[...]) + b2_ref[...]
    z_ref[...] = z
    a_ref[...] = jnp.dot(z, sw1a_ref[...], **_PREC) + sb1_ref[...]
    bb_ref[...] = jnp.dot(z, sw1b_ref[...], **_PREC)
    t = jnp.dot(z, fw1_ref[...], **_PREC) + fb1_ref[...]
    f = jnp.where(t > 0, t, 0.1 * t)
    fr_ref[...] = jnp.dot(f, fw2_ref[...], **_PREC) + fb2_ref[...]
    dh = jnp.maximum(jnp.dot(z, dw1_ref[...], **_PREC) + db1_ref[...], 0.0)
    pd_ref[...] = jnp.sum(dh * dw2t_ref[...], axis=1, keepdims=True) \
        + db2_ref[...]


def _mm3(acc2, y2, dis, b2, sW1a, sW1b, sb1, fW1, fb1, fW2, fb2, dW1, db1,
         dW2t, db2):
    R = 1024
    full = lambda shp: pl.BlockSpec(shp, lambda i: tuple(0 for _ in shp))
    return pl.pallas_call(
        _mm3_body,
        grid=(NPAD // R,),
        in_specs=[
            pl.BlockSpec((NC, R, H), lambda i: (0, i, 0)),
            pl.BlockSpec((R, H), lambda i: (i, 0)),
            pl.BlockSpec((R, 1), lambda i: (i, 0)),
            full((1, H)),
            full((H, H)),
            full((H, H)),
            full((1, H)),
            full((H, H // 2)),
            full((1, H // 2)),
            full((H // 2, D)),
            full((1, D)),
            full((H, H)),
            full((1, H)),
            full((1, H)),
            full((1, 1)),
        ],
        out_specs=[
            pl.BlockSpec((R, H), lambda i: (i, 0)),
            pl.BlockSpec((R, H), lambda i: (i, 0)),
            pl.BlockSpec((R, H), lambda i: (i, 0)),
            pl.BlockSpec((R, D), lambda i: (i, 0)),
            pl.BlockSpec((R, 1), lambda i: (i, 0)),
        ],
        out_shape=[
            jax.ShapeDtypeStruct((NPAD, H), jnp.float32),
            jax.ShapeDtypeStruct((NPAD, H), jnp.float32),
            jax.ShapeDtypeStruct((NPAD, H), jnp.float32),
            jax.ShapeDtypeStruct((NPAD, D), jnp.float32),
            jax.ShapeDtypeStruct((NPAD, 1), jnp.float32),
        ],
    )(acc2, y2, dis, b2, sW1a, sW1b, sb1, fW1, fb1, fW2, fb2, dW1, db1,
      dW2t, db2)


def kernel(x, edge_index, batch, W1, b1, W2, b2, fW1, fb1, fW2, fb2,
           sW1, sb1, sW2, sb2, dW1, db1, dW2, db2):
    src = edge_index[0]
    dst = edge_index[1]
    pad = N + (jnp.arange(EP - E, dtype=jnp.int32) % (NPAD - N))
    src3 = jnp.concatenate([src, pad]).reshape(NW, NCHUNK, CH)
    dst3 = jnp.concatenate([dst, pad]).reshape(NW, NCHUNK, CH)
    srcp = src3.reshape(NW, PNCHUNK, PCH)
    dstp = dst3.reshape(NW, PNCHUNK, PCH)
    xp = jnp.pad(x, ((0, NPAD - N), (0, 0)))
    zerosH = jnp.zeros((NPAD, H), jnp.float32)

    degp = _make_deg()(dst3)
    deg2 = (degp[0] + degp[1] + 1.0)[:, None]

    y1, dis = _mm1(xp, W1, deg2)
    acc1 = _make_prop()(y1, srcp, dstp, zerosH)
    y2 = _mm2(acc1, y1, dis, b1.reshape(1, H), W2)
    acc2 = _make_prop()(y2, srcp, dstp, zerosH)

    z, A, B, fr, pd = _mm3(
        acc2, y2, dis, b2.reshape(1, H), sW1[:H], sW1[H:], sb1.reshape(1, H),
        fW1, fb1.reshape(1, H // 2), fW2, fb2.reshape(1, D),
        dW1, db1.reshape(1, H), dW2.reshape(1, H), db2.reshape(1, 1))
    sb2v = jnp.full((16,), sb2[0], jnp.float32)
    sr = _make_struct()(A, B, src3, dst3, sW2.reshape(H), sb2v)

    sr_flat = sr[:, :EPW].reshape(EP)[:E]
    return (z[:N], fr[:N], sr_flat.reshape(E, 1), pd[:N])
